# baseline (speedup 1.0000x reference)
import functools

import jax
import jax.numpy as jnp
from jax import lax
from jax.experimental import pallas as pl
from jax.experimental.pallas import tpu as pltpu
from jax.experimental.pallas import tpu_sc as plsc

_NC = 2
_NS = 16
_NW = _NC * _NS
_BPC = 4
_NBUF = 4
_DEPTH = 2
_PAD = 128


@functools.cache
def _build(batch, n_fields, D, V):
    b_per_w = batch // _NW
    n_chunks = b_per_w // _BPC
    rows_per_chunk = _BPC * n_fields
    mesh = plsc.VectorSubcoreMesh(core_axis_name="c", subcore_axis_name="s")

    @functools.partial(
        pl.kernel,
        mesh=mesh,
        out_type=jax.ShapeDtypeStruct((batch, n_fields, D), jnp.float32),
        scratch_types=[
            pltpu.VMEM((n_chunks, rows_per_chunk), jnp.int32),
            [pltpu.VMEM((rows_per_chunk, _PAD), jnp.float32)
             for _ in range(_NBUF)],
            [pltpu.VMEM((rows_per_chunk, D), jnp.float32)
             for _ in range(_NBUF)],
            [pltpu.SemaphoreType.DMA for _ in range(_NBUF)],
            [pltpu.SemaphoreType.DMA for _ in range(_NBUF)],
        ],
    )
    def gather_kernel(idx_hbm, dense_hbm, out_hbm, idx_v, rows, packed,
                      gsem, ssem):
        sid = lax.axis_index("s")
        wid = sid * _NC + lax.axis_index("c")
        b0 = wid * b_per_w
        pltpu.sync_copy(idx_hbm.at[pl.ds(wid * n_chunks, n_chunks)], idx_v)

        def start_gather(c):
            b = c % _NBUF
            return pltpu.async_copy(
                dense_hbm.at[idx_v.at[c]], rows[b], gsem[b])

        def do_pack(c):
            b = c % _NBUF

            def body(i, carry):
                r0 = i * 8
                for rr in range(8):
                    for g in range(D // 16):
                        packed[b][r0 + rr, pl.ds(g * 16, 16)] = (
                            rows[b][r0 + rr, pl.ds(g * 16, 16)])
                return carry

            lax.fori_loop(0, rows_per_chunk // 8, body, 0)

        def start_store(c):
            b = c % _NBUF
            return [
                pltpu.async_copy(
                    packed[b].at[pl.ds(j * n_fields, n_fields)],
                    out_hbm.at[b0 + c * _BPC + j], ssem[b])
                for j in range(_BPC)
            ]

        gathers = [None] * n_chunks
        stores = [None] * n_chunks
        for j in range(min(_DEPTH, n_chunks)):
            gathers[j] = start_gather(j)
        for t in range(n_chunks + 1):
            c_p = t
            if c_p < n_chunks:
                gathers[c_p].wait()
                if c_p >= _NBUF:
                    for d in stores[c_p - _NBUF]:
                        d.wait()
                do_pack(c_p)
                stores[c_p] = start_store(c_p)
            c_g = t + _DEPTH
            if c_g < n_chunks:
                gathers[c_g] = start_gather(c_g)
        for c in range(max(0, n_chunks - _NBUF), n_chunks):
            for d in stores[c]:
                d.wait()

    return gather_kernel


def kernel(indices, weight):
    batch, n_fields = indices.shape
    V, D = weight.shape
    idx2 = indices.reshape(batch // _BPC, _BPC * n_fields)
    dense = jnp.pad(weight, ((0, 0), (0, _PAD - D)))
    return _build(batch, n_fields, D, V)(idx2, dense)


# speedup vs baseline: 1.0193x; 1.0193x over previous
import functools

import jax
import jax.numpy as jnp
from jax import lax
from jax.experimental import pallas as pl
from jax.experimental.pallas import tpu as pltpu
from jax.experimental.pallas import tpu_sc as plsc

_NC = 2
_NS = 16
_NW = _NC * _NS
_BPC = 4
_NBUF = 4
_DEPTH = 2
_PAD = 128


@functools.cache
def _build(batch, n_fields, D, V):
    b_per_w = batch // _NW
    n_chunks = b_per_w // _BPC
    rows_per_chunk = _BPC * n_fields
    mesh = plsc.VectorSubcoreMesh(core_axis_name="c", subcore_axis_name="s")

    @functools.partial(
        pl.kernel,
        mesh=mesh,
        out_type=jax.ShapeDtypeStruct((batch, n_fields, D), jnp.float32),
        scratch_types=[
            pltpu.VMEM((n_chunks, rows_per_chunk), jnp.int32),
            [pltpu.VMEM((rows_per_chunk, _PAD), jnp.float32)
             for _ in range(_NBUF)],
            [pltpu.VMEM((rows_per_chunk, D), jnp.float32)
             for _ in range(_NBUF)],
            [pltpu.SemaphoreType.DMA for _ in range(_NBUF)],
            [pltpu.SemaphoreType.DMA for _ in range(_NBUF)],
        ],
    )
    def gather_kernel(idx_hbm, dense_hbm, out_hbm, idx_v, rows, packed,
                      gsem, ssem):
        sid = lax.axis_index("s")
        wid = sid * _NC + lax.axis_index("c")
        b0 = wid * b_per_w
        pltpu.sync_copy(idx_hbm.at[pl.ds(wid * n_chunks, n_chunks)], idx_v)

        def start_gather(c):
            b = c % _NBUF
            return pltpu.async_copy(
                dense_hbm.at[idx_v.at[c]], rows[b], gsem[b])

        def do_pack(c):
            b = c % _NBUF

            def body(r, carry):
                for g in range(D // 16):
                    packed[b][r, pl.ds(g * 16, 16)] = (
                        rows[b][r, pl.ds(g * 16, 16)])
                return carry

            lax.fori_loop(0, rows_per_chunk, body, 0)

        def start_store(c):
            b = c % _NBUF
            return [
                pltpu.async_copy(
                    packed[b].at[pl.ds(j * n_fields, n_fields)],
                    out_hbm.at[b0 + c * _BPC + j], ssem[b])
                for j in range(_BPC)
            ]

        gathers = [None] * n_chunks
        stores = [None] * n_chunks
        for j in range(min(_DEPTH, n_chunks)):
            gathers[j] = start_gather(j)
        for t in range(n_chunks + 1):
            c_p = t
            if c_p < n_chunks:
                gathers[c_p].wait()
                if c_p >= _NBUF:
                    for d in stores[c_p - _NBUF]:
                        d.wait()
                do_pack(c_p)
                stores[c_p] = start_store(c_p)
            c_g = t + _DEPTH
            if c_g < n_chunks:
                gathers[c_g] = start_gather(c_g)
        for c in range(max(0, n_chunks - _NBUF), n_chunks):
            for d in stores[c]:
                d.wait()

    return gather_kernel


def kernel(indices, weight):
    batch, n_fields = indices.shape
    V, D = weight.shape
    idx2 = indices.reshape(batch // _BPC, _BPC * n_fields)
    dense = jnp.pad(weight, ((0, 0), (0, _PAD - D)))
    return _build(batch, n_fields, D, V)(idx2, dense)


# speedup vs baseline: 1.0312x; 1.0117x over previous
import functools

import jax
import jax.numpy as jnp
from jax import lax
from jax.experimental import pallas as pl
from jax.experimental.pallas import tpu as pltpu
from jax.experimental.pallas import tpu_sc as plsc

_NC = 2
_NS = 16
_NW = _NC * _NS
_BPC = 4
_NBUF = 4
_DEPTH = 3
_PAD = 128


@functools.cache
def _build(batch, n_fields, D, V):
    b_per_w = batch // _NW
    n_chunks = b_per_w // _BPC
    rows_per_chunk = _BPC * n_fields
    mesh = plsc.VectorSubcoreMesh(core_axis_name="c", subcore_axis_name="s")

    @functools.partial(
        pl.kernel,
        mesh=mesh,
        out_type=jax.ShapeDtypeStruct((batch, n_fields, D), jnp.float32),
        scratch_types=[
            pltpu.VMEM((n_chunks, rows_per_chunk), jnp.int32),
            [pltpu.VMEM((rows_per_chunk, _PAD), jnp.float32)
             for _ in range(_NBUF)],
            [pltpu.VMEM((rows_per_chunk, D), jnp.float32)
             for _ in range(_NBUF)],
            [pltpu.SemaphoreType.DMA for _ in range(_NBUF)],
            [pltpu.SemaphoreType.DMA for _ in range(_NBUF)],
        ],
    )
    def gather_kernel(idx_hbm, dense_hbm, out_hbm, idx_v, rows, packed,
                      gsem, ssem):
        sid = lax.axis_index("s")
        wid = sid * _NC + lax.axis_index("c")
        b0 = wid * b_per_w
        pltpu.sync_copy(idx_hbm.at[pl.ds(wid * n_chunks, n_chunks)], idx_v)

        def start_gather(c):
            b = c % _NBUF
            return pltpu.async_copy(
                dense_hbm.at[idx_v.at[c]], rows[b], gsem[b])

        def do_pack(c):
            b = c % _NBUF

            def body(r, carry):
                for g in range(D // 16):
                    packed[b][r, pl.ds(g * 16, 16)] = (
                        rows[b][r, pl.ds(g * 16, 16)])
                return carry

            lax.fori_loop(0, rows_per_chunk, body, 0)

        def start_store(c):
            b = c % _NBUF
            return [
                pltpu.async_copy(
                    packed[b].at[pl.ds(j * n_fields, n_fields)],
                    out_hbm.at[b0 + c * _BPC + j], ssem[b])
                for j in range(_BPC)
            ]

        gathers = [None] * n_chunks
        stores = [None] * n_chunks
        for j in range(min(_DEPTH, n_chunks)):
            gathers[j] = start_gather(j)
        for t in range(n_chunks + 1):
            c_p = t
            if c_p < n_chunks:
                gathers[c_p].wait()
                if c_p >= _NBUF:
                    for d in stores[c_p - _NBUF]:
                        d.wait()
                do_pack(c_p)
                stores[c_p] = start_store(c_p)
            c_g = t + _DEPTH
            if c_g < n_chunks:
                gathers[c_g] = start_gather(c_g)
        for c in range(max(0, n_chunks - _NBUF), n_chunks):
            for d in stores[c]:
                d.wait()

    return gather_kernel


def kernel(indices, weight):
    batch, n_fields = indices.shape
    V, D = weight.shape
    idx2 = indices.reshape(batch // _BPC, _BPC * n_fields)
    dense = jnp.pad(weight, ((0, 0), (0, _PAD - D)))
    return _build(batch, n_fields, D, V)(idx2, dense)
